# baseline (speedup 1.0000x reference)
import dataclasses
import functools

import jax
import jax.numpy as jnp
from jax import lax
from jax.experimental import pallas as pl
from jax.experimental.pallas import tpu as pltpu
from jax.experimental.pallas import tpu_sc as plsc

_N = 10000
_NR = 10112
_HALF = _NR // 2
_RPTA = _NR // 16
_CH = 128
_CPT_DEG = 80
_CPT_AGG = 160
_KB = 2
_E_PAD = 32 * _CPT_DEG * _CH
_BN = 632
_PAD_DST = 10008

_sc_mesh = plsc.VectorSubcoreMesh(core_axis_name="c", subcore_axis_name="s")

_sc_params = pltpu.CompilerParams(
    needs_layout_passes=False, use_tc_tiling_on_sc=False)


def _make_sc_degree():
    @functools.partial(
        pl.kernel,
        out_type=jax.ShapeDtypeStruct((4, _NR, 128), jnp.float32),
        mesh=_sc_mesh,
        compiler_params=_sc_params,
        scratch_types=[
            pltpu.VMEM((_CPT_DEG, _CH), jnp.int32),
            pltpu.VMEM((_HALF, 16), jnp.float32),
        ],
    )
    def deg_kernel(dst_hbm, out_hbm, idx_v, hist_v):
        cid = lax.axis_index("c")
        sid = lax.axis_index("s")
        wid = sid * 2 + cid
        zero16 = jnp.zeros((16,), jnp.float32)
        one16 = jnp.full((16,), 1.0, jnp.float32)
        lane16 = lax.iota(jnp.int32, 16)

        pltpu.sync_copy(dst_hbm.at[pl.ds(wid * _CPT_DEG, _CPT_DEG)], idx_v)

        for h in range(2):
            lo = h * _HALF

            @pl.loop(0, _HALF)
            def _(r):
                hist_v[r, :] = zero16

            @pl.loop(0, _CPT_DEG)
            def _(c):
                for j in range(_CH // 16):
                    d16 = idx_v[c, pl.ds(16 * j, 16)]
                    m = (d16 >= lo) & (d16 < lo + _HALF)
                    plsc.addupdate_scatter(hist_v, [d16 - lo, lane16],
                                           one16, mask=m)

            pltpu.sync_copy(hist_v,
                            out_hbm.at[wid // 8, pl.ds(lo, _HALF),
                                       pl.ds(16 * (wid % 8), 16)])

    return deg_kernel


def _make_sc_agg(d: int, kb: int):
    nblk = _CPT_AGG // kb

    @functools.partial(
        pl.kernel,
        out_type=jax.ShapeDtypeStruct((2, _NR, d), jnp.float32),
        mesh=_sc_mesh,
        compiler_params=_sc_params,
        scratch_types=[
            pltpu.VMEM((nblk, kb * _CH), jnp.int32),
            pltpu.VMEM((nblk, kb * _CH), jnp.int32),
            pltpu.VMEM((kb * _CH, d), jnp.float32),
            pltpu.VMEM((kb * _CH, d), jnp.float32),
            pltpu.VMEM_SHARED((_NR, d), jnp.float32),
            pltpu.SemaphoreType.DMA,
            pltpu.SemaphoreType.DMA,
            pltpu.SemaphoreType.DMA,
            pltpu.SemaphoreType.DMA,
        ],
    )
    def agg_kernel(y_hbm, src_hbm, dst_hbm, out_hbm,
                   src_v, dst_v, rows_v, rows1_v, acc_sh,
                   sg0, sg1, ss0, ss1):
        cid = lax.axis_index("c")
        sid = lax.axis_index("s")
        zero16 = jnp.zeros((16,), jnp.float32)

        pltpu.sync_copy(src_hbm.at[pl.ds(sid * nblk, nblk)], src_v)
        pltpu.sync_copy(dst_hbm.at[pl.ds(sid * nblk, nblk)], dst_v)

        @pl.loop(0, min(_RPTA, kb * _CH))
        def _(r):
            for j in range(d // 16):
                rows_v[r, pl.ds(16 * j, 16)] = zero16

        _nz = kb * _CH
        _off = 0
        while _off < _RPTA:
            _m = min(_nz, _RPTA - _off)
            pltpu.sync_copy(
                rows_v.at[pl.ds(0, _m)],
                acc_sh.at[pl.ds(sid * _RPTA + _off, _m)])
            _off += _m
        plsc.subcore_barrier()

        def _g_start(c, buf, sem):
            pltpu.make_async_copy(
                y_hbm.at[cid].at[src_v.at[c]], buf, sem).start()

        def _g_wait(c, buf, sem):
            pltpu.make_async_copy(
                y_hbm.at[cid].at[src_v.at[c]], buf, sem).wait()

        def _s_start(c, buf, sem):
            pltpu.async_copy(buf, acc_sh.at[dst_v.at[c]], sem, add=True)

        def _s_wait(c, buf, sem):
            pltpu.make_async_copy(buf, acc_sh.at[dst_v.at[c]], sem).wait()

        npair = nblk // 2
        _g_start(0, rows_v, sg0)

        @pl.loop(0, npair)
        def _(p):
            c0 = 2 * p
            _g_start(c0 + 1, rows1_v, sg1)
            _g_wait(c0, rows_v, sg0)
            pltpu.sync_copy(rows_v, acc_sh.at[dst_v.at[c0]], add=True)

            @pl.when(p + 1 < npair)
            def _():
                _g_start(c0 + 2, rows_v, sg0)

            _g_wait(c0 + 1, rows1_v, sg1)
            pltpu.sync_copy(rows1_v, acc_sh.at[dst_v.at[c0 + 1]], add=True)

        plsc.subcore_barrier()
        pltpu.sync_copy(acc_sh.at[pl.ds(sid * _RPTA, _RPTA)],
                        out_hbm.at[cid, pl.ds(sid * _RPTA, _RPTA)])

    return agg_kernel


_sc_degree = _make_sc_degree()
_sc_agg64 = _make_sc_agg(64, _KB)
_sc_agg32 = _make_sc_agg(32, _KB)


def _tc_first(dp, h, w1):
    n, din = h.shape
    dh = w1.shape[1] // 2
    g = n // _BN

    def body(dp_ref, h_ref, w_ref, y_ref, norm_ref):
        deg = jnp.sum(dp_ref[...], axis=(0, 2))[:, None] + 1.0
        nrm = lax.rsqrt(jnp.maximum(deg, 1.0))
        norm_ref[...] = nrm
        y = jnp.dot(h_ref[...] * nrm, w_ref[...],
                    preferred_element_type=jnp.float32)
        y_ref[0] = y[:, :dh]
        y_ref[1] = y[:, dh:]

    return pl.pallas_call(
        body,
        grid=(g,),
        in_specs=[
            pl.BlockSpec((4, _BN, 128), lambda i: (0, i, 0)),
            pl.BlockSpec((_BN, din), lambda i: (i, 0)),
            pl.BlockSpec((din, 2 * dh), lambda i: (0, 0)),
        ],
        out_specs=[
            pl.BlockSpec((2, _BN, dh), lambda i: (0, i, 0)),
            pl.BlockSpec((_BN, 1), lambda i: (i, 0)),
        ],
        out_shape=[
            jax.ShapeDtypeStruct((2, n, dh), jnp.float32),
            jax.ShapeDtypeStruct((n, 1), jnp.float32),
        ],
    )(dp, h, w1)


def _tc_mid(p, y, norm, b, w):
    n = y.shape[1]
    d = 2 * y.shape[2]
    dout = w.shape[1]
    g = n // _BN

    def body(p_ref, y_ref, n_ref, b_ref, w_ref, o_ref):
        agg = jnp.concatenate(
            [p_ref[0] + y_ref[0], p_ref[1] + y_ref[1]], axis=-1)
        x = jnp.maximum(n_ref[...] * agg + b_ref[...], 0.0)
        yn = jnp.dot(x * n_ref[...], w_ref[...],
                     preferred_element_type=jnp.float32)
        o_ref[0] = yn[:, :dout // 2]
        o_ref[1] = yn[:, dout // 2:]

    return pl.pallas_call(
        body,
        grid=(g,),
        in_specs=[
            pl.BlockSpec((2, _BN, d // 2), lambda i: (0, i, 0)),
            pl.BlockSpec((2, _BN, d // 2), lambda i: (0, i, 0)),
            pl.BlockSpec((_BN, 1), lambda i: (i, 0)),
            pl.BlockSpec((1, d), lambda i: (0, 0)),
            pl.BlockSpec((d, dout), lambda i: (0, 0)),
        ],
        out_specs=pl.BlockSpec((2, _BN, dout // 2), lambda i: (0, i, 0)),
        out_shape=jax.ShapeDtypeStruct((2, n, dout // 2), jnp.float32),
    )(p, y, norm, b, w)


def _tc_last(p, y, norm, b):
    n = y.shape[1]
    d = 2 * y.shape[2]
    g = n // _BN

    def body(p_ref, y_ref, n_ref, b_ref, o_ref):
        agg = jnp.concatenate(
            [p_ref[0] + y_ref[0], p_ref[1] + y_ref[1]], axis=-1)
        o_ref[...] = n_ref[...] * agg + b_ref[...]

    return pl.pallas_call(
        body,
        grid=(g,),
        in_specs=[
            pl.BlockSpec((2, _BN, d // 2), lambda i: (0, i, 0)),
            pl.BlockSpec((2, _BN, d // 2), lambda i: (0, i, 0)),
            pl.BlockSpec((_BN, 1), lambda i: (i, 0)),
            pl.BlockSpec((1, d), lambda i: (0, 0)),
        ],
        out_specs=pl.BlockSpec((_BN, d), lambda i: (i, 0)),
        out_shape=jax.ShapeDtypeStruct((n, d), jnp.float32),
    )(p, y, norm, b)


def kernel(h, edge_index, W1, b1, W2, b2, W3, b3):
    n = h.shape[0]
    src = edge_index[0]
    dst = edge_index[1]
    pad = _E_PAD - src.shape[0]
    src_p = jnp.concatenate([src, jnp.zeros((pad,), jnp.int32)])
    dst_p = jnp.concatenate([dst, jnp.full((pad,), _PAD_DST, jnp.int32)])
    srcA = src_p.reshape(_E_PAD // (_KB * _CH), _KB * _CH)
    dstA = dst_p.reshape(_E_PAD // (_KB * _CH), _KB * _CH)
    srcB = src_p.reshape(_E_PAD // (2 * _KB * _CH), 2 * _KB * _CH)
    dstB = dst_p.reshape(_E_PAD // (2 * _KB * _CH), 2 * _KB * _CH)

    hp = jnp.pad(h, ((0, _NR - n), (0, 0)))

    degp = _sc_degree(dst_p.reshape(_E_PAD // _CH, _CH))

    y1, norm = _tc_first(degp, hp, W1)
    p1 = _sc_agg64(y1, srcA, dstA)
    y2 = _tc_mid(p1, y1, norm, b1.reshape(1, -1), W2)
    p2 = _sc_agg64(y2, srcA, dstA)
    y3 = _tc_mid(p2, y2, norm, b2.reshape(1, -1), W3)
    p3 = _sc_agg32(y3, srcA, dstA)
    return _tc_last(p3, y3, norm, b3.reshape(1, -1))[:n]


# speedup vs baseline: 1.0192x; 1.0192x over previous
import dataclasses
import functools

import jax
import jax.numpy as jnp
from jax import lax
from jax.experimental import pallas as pl
from jax.experimental.pallas import tpu as pltpu
from jax.experimental.pallas import tpu_sc as plsc

_N = 10000
_NR = 10112
_HALF = _NR // 2
_RPTA = _NR // 16
_CH = 128
_CPT_DEG = 80
_CPT_AGG = 160
_KB = 2
_E_PAD = 32 * _CPT_DEG * _CH
_BN = 632
_PAD_DST = 10008

_sc_mesh = plsc.VectorSubcoreMesh(core_axis_name="c", subcore_axis_name="s")

_sc_params = pltpu.CompilerParams(
    needs_layout_passes=False, use_tc_tiling_on_sc=False)


def _make_sc_degree():
    @functools.partial(
        pl.kernel,
        out_type=jax.ShapeDtypeStruct((_NR, 512), jnp.float32),
        mesh=_sc_mesh,
        compiler_params=_sc_params,
        scratch_types=[
            pltpu.VMEM((_CPT_DEG, _CH), jnp.int32),
            pltpu.VMEM((_HALF, 16), jnp.float32),
        ],
    )
    def deg_kernel(dst_hbm, out_hbm, idx_v, hist_v):
        cid = lax.axis_index("c")
        sid = lax.axis_index("s")
        wid = sid * 2 + cid
        zero16 = jnp.zeros((16,), jnp.float32)
        one16 = jnp.full((16,), 1.0, jnp.float32)
        lane16 = lax.iota(jnp.int32, 16)

        pltpu.sync_copy(dst_hbm.at[pl.ds(wid * _CPT_DEG, _CPT_DEG)], idx_v)

        for h in range(2):
            lo = h * _HALF

            @pl.loop(0, _HALF)
            def _(r):
                hist_v[r, :] = zero16

            @pl.loop(0, _CPT_DEG)
            def _(c):
                for j in range(_CH // 16):
                    d16 = idx_v[c, pl.ds(16 * j, 16)]
                    m = (d16 >= lo) & (d16 < lo + _HALF)
                    plsc.addupdate_scatter(hist_v, [d16 - lo, lane16],
                                           one16, mask=m)

            pltpu.sync_copy(hist_v,
                            out_hbm.at[pl.ds(lo, _HALF),
                                       pl.ds(16 * wid, 16)])

    return deg_kernel


def _make_sc_agg(d: int, kb: int):
    nblk = _CPT_AGG // kb

    @functools.partial(
        pl.kernel,
        out_type=jax.ShapeDtypeStruct((2, _NR, d), jnp.float32),
        mesh=_sc_mesh,
        compiler_params=_sc_params,
        scratch_types=[
            pltpu.VMEM((nblk, kb * _CH), jnp.int32),
            pltpu.VMEM((nblk, kb * _CH), jnp.int32),
            pltpu.VMEM((kb * _CH, d), jnp.float32),
            pltpu.VMEM((kb * _CH, d), jnp.float32),
            pltpu.VMEM_SHARED((_NR, d), jnp.float32),
            pltpu.SemaphoreType.DMA,
            pltpu.SemaphoreType.DMA,
            pltpu.SemaphoreType.DMA,
            pltpu.SemaphoreType.DMA,
        ],
    )
    def agg_kernel(y_hbm, src_hbm, dst_hbm, out_hbm,
                   src_v, dst_v, rows_v, rows1_v, acc_sh,
                   sg0, sg1, ss0, ss1):
        cid = lax.axis_index("c")
        sid = lax.axis_index("s")
        zero16 = jnp.zeros((16,), jnp.float32)

        pltpu.sync_copy(src_hbm.at[pl.ds(sid * nblk, nblk)], src_v)
        pltpu.sync_copy(dst_hbm.at[pl.ds(sid * nblk, nblk)], dst_v)

        @pl.loop(0, min(_RPTA, kb * _CH))
        def _(r):
            for j in range(d // 16):
                rows_v[r, pl.ds(16 * j, 16)] = zero16

        _nz = kb * _CH
        _off = 0
        while _off < _RPTA:
            _m = min(_nz, _RPTA - _off)
            pltpu.sync_copy(
                rows_v.at[pl.ds(0, _m)],
                acc_sh.at[pl.ds(sid * _RPTA + _off, _m)])
            _off += _m
        plsc.subcore_barrier()

        def _g_start(c, buf, sem):
            pltpu.make_async_copy(
                y_hbm.at[cid].at[src_v.at[c]], buf, sem).start()

        def _g_wait(c, buf, sem):
            pltpu.make_async_copy(
                y_hbm.at[cid].at[src_v.at[c]], buf, sem).wait()

        def _s_start(c, buf, sem):
            pltpu.async_copy(buf, acc_sh.at[dst_v.at[c]], sem, add=True)

        def _s_wait(c, buf, sem):
            pltpu.make_async_copy(buf, acc_sh.at[dst_v.at[c]], sem).wait()

        npair = nblk // 2
        _g_start(0, rows_v, sg0)

        @pl.loop(0, npair)
        def _(p):
            c0 = 2 * p
            _g_start(c0 + 1, rows1_v, sg1)
            _g_wait(c0, rows_v, sg0)
            pltpu.sync_copy(rows_v, acc_sh.at[dst_v.at[c0]], add=True)

            @pl.when(p + 1 < npair)
            def _():
                _g_start(c0 + 2, rows_v, sg0)

            _g_wait(c0 + 1, rows1_v, sg1)
            pltpu.sync_copy(rows1_v, acc_sh.at[dst_v.at[c0 + 1]], add=True)

        plsc.subcore_barrier()
        pltpu.sync_copy(acc_sh.at[pl.ds(sid * _RPTA, _RPTA)],
                        out_hbm.at[cid, pl.ds(sid * _RPTA, _RPTA)])

    return agg_kernel


_sc_degree = _make_sc_degree()
_sc_agg64 = _make_sc_agg(64, _KB)
_sc_agg32 = _make_sc_agg(32, _KB)


def _tc_first(dp, h, w1):
    n, din = h.shape
    dh = w1.shape[1] // 2
    g = n // _BN

    def body(dp_ref, h_ref, w_ref, y_ref, norm_ref):
        deg = jnp.sum(dp_ref[...], axis=1)[:, None] + 1.0
        nrm = lax.rsqrt(jnp.maximum(deg, 1.0))
        norm_ref[...] = nrm
        y = jnp.dot(h_ref[...] * nrm, w_ref[...],
                    preferred_element_type=jnp.float32)
        y_ref[0] = y[:, :dh]
        y_ref[1] = y[:, dh:]

    return pl.pallas_call(
        body,
        grid=(g,),
        in_specs=[
            pl.BlockSpec((_BN, 512), lambda i: (i, 0)),
            pl.BlockSpec((_BN, din), lambda i: (i, 0)),
            pl.BlockSpec((din, 2 * dh), lambda i: (0, 0)),
        ],
        out_specs=[
            pl.BlockSpec((2, _BN, dh), lambda i: (0, i, 0)),
            pl.BlockSpec((_BN, 1), lambda i: (i, 0)),
        ],
        out_shape=[
            jax.ShapeDtypeStruct((2, n, dh), jnp.float32),
            jax.ShapeDtypeStruct((n, 1), jnp.float32),
        ],
    )(dp, h, w1)


def _tc_mid(p, y, norm, b, w):
    n = y.shape[1]
    d = 2 * y.shape[2]
    dout = w.shape[1]
    g = n // _BN

    def body(p_ref, y_ref, n_ref, b_ref, w_ref, o_ref):
        agg = jnp.concatenate(
            [p_ref[0] + y_ref[0], p_ref[1] + y_ref[1]], axis=-1)
        x = jnp.maximum(n_ref[...] * agg + b_ref[...], 0.0)
        yn = jnp.dot(x * n_ref[...], w_ref[...],
                     preferred_element_type=jnp.float32)
        o_ref[0] = yn[:, :dout // 2]
        o_ref[1] = yn[:, dout // 2:]

    return pl.pallas_call(
        body,
        grid=(g,),
        in_specs=[
            pl.BlockSpec((2, _BN, d // 2), lambda i: (0, i, 0)),
            pl.BlockSpec((2, _BN, d // 2), lambda i: (0, i, 0)),
            pl.BlockSpec((_BN, 1), lambda i: (i, 0)),
            pl.BlockSpec((1, d), lambda i: (0, 0)),
            pl.BlockSpec((d, dout), lambda i: (0, 0)),
        ],
        out_specs=pl.BlockSpec((2, _BN, dout // 2), lambda i: (0, i, 0)),
        out_shape=jax.ShapeDtypeStruct((2, n, dout // 2), jnp.float32),
    )(p, y, norm, b, w)


def _tc_last(p, y, norm, b):
    n = y.shape[1]
    d = 2 * y.shape[2]
    g = n // _BN

    def body(p_ref, y_ref, n_ref, b_ref, o_ref):
        agg = jnp.concatenate(
            [p_ref[0] + y_ref[0], p_ref[1] + y_ref[1]], axis=-1)
        o_ref[...] = n_ref[...] * agg + b_ref[...]

    return pl.pallas_call(
        body,
        grid=(g,),
        in_specs=[
            pl.BlockSpec((2, _BN, d // 2), lambda i: (0, i, 0)),
            pl.BlockSpec((2, _BN, d // 2), lambda i: (0, i, 0)),
            pl.BlockSpec((_BN, 1), lambda i: (i, 0)),
            pl.BlockSpec((1, d), lambda i: (0, 0)),
        ],
        out_specs=pl.BlockSpec((_BN, d), lambda i: (i, 0)),
        out_shape=jax.ShapeDtypeStruct((n, d), jnp.float32),
    )(p, y, norm, b)


def kernel(h, edge_index, W1, b1, W2, b2, W3, b3):
    n = h.shape[0]
    src = edge_index[0]
    dst = edge_index[1]
    pad = _E_PAD - src.shape[0]
    src_p = jnp.concatenate([src, jnp.zeros((pad,), jnp.int32)])
    dst_p = jnp.concatenate([dst, jnp.full((pad,), _PAD_DST, jnp.int32)])
    srcA = src_p.reshape(_E_PAD // (_KB * _CH), _KB * _CH)
    dstA = dst_p.reshape(_E_PAD // (_KB * _CH), _KB * _CH)
    srcB = src_p.reshape(_E_PAD // (2 * _KB * _CH), 2 * _KB * _CH)
    dstB = dst_p.reshape(_E_PAD // (2 * _KB * _CH), 2 * _KB * _CH)

    hp = jnp.pad(h, ((0, _NR - n), (0, 0)))

    degp = _sc_degree(dst_p.reshape(_E_PAD // _CH, _CH))

    y1, norm = _tc_first(degp, hp, W1)
    p1 = _sc_agg64(y1, srcA, dstA)
    y2 = _tc_mid(p1, y1, norm, b1.reshape(1, -1), W2)
    p2 = _sc_agg64(y2, srcA, dstA)
    y3 = _tc_mid(p2, y2, norm, b2.reshape(1, -1), W3)
    p3 = _sc_agg32(y3, srcA, dstA)
    return _tc_last(p3, y3, norm, b3.reshape(1, -1))[:n]


# speedup vs baseline: 1.0199x; 1.0007x over previous
import functools

import jax
import jax.numpy as jnp
from jax import lax
from jax.experimental import pallas as pl
from jax.experimental.pallas import tpu as pltpu
from jax.experimental.pallas import tpu_sc as plsc

_N = 10000
_NR = 10112
_HALF = _NR // 2
_RPTA = _NR // 16
_CH = 128
_CPT_DEG = 80
_CPT_AGG = 160
_KB = 2
_E_PAD = 32 * _CPT_DEG * _CH
_BN = 632
_PAD_DST = 10008

_sc_mesh = plsc.VectorSubcoreMesh(core_axis_name="c", subcore_axis_name="s")

_sc_params = pltpu.CompilerParams(
    needs_layout_passes=False, use_tc_tiling_on_sc=False)


def _make_sc_degree():
    @functools.partial(
        pl.kernel,
        out_type=jax.ShapeDtypeStruct((_NR, 512), jnp.float32),
        mesh=_sc_mesh,
        compiler_params=_sc_params,
        scratch_types=[
            pltpu.VMEM((_CPT_DEG, _CH), jnp.int32),
            pltpu.VMEM((_HALF, 16), jnp.float32),
        ],
    )
    def deg_kernel(dst_hbm, out_hbm, idx_v, hist_v):
        cid = lax.axis_index("c")
        sid = lax.axis_index("s")
        wid = sid * 2 + cid
        zero16 = jnp.zeros((16,), jnp.float32)
        one16 = jnp.full((16,), 1.0, jnp.float32)
        lane16 = lax.iota(jnp.int32, 16)

        pltpu.sync_copy(dst_hbm.at[pl.ds(wid * _CPT_DEG, _CPT_DEG)], idx_v)

        for h in range(2):
            lo = h * _HALF

            @pl.loop(0, _HALF)
            def _(r):
                hist_v[r, :] = zero16

            @pl.loop(0, _CPT_DEG)
            def _(c):
                for j in range(_CH // 16):
                    d16 = idx_v[c, pl.ds(16 * j, 16)]
                    m = (d16 >= lo) & (d16 < lo + _HALF)
                    plsc.addupdate_scatter(hist_v, [d16 - lo, lane16],
                                           one16, mask=m)

            pltpu.sync_copy(hist_v,
                            out_hbm.at[pl.ds(lo, _HALF),
                                       pl.ds(16 * wid, 16)])

    return deg_kernel


def _make_sc_agg(d: int, kb: int):
    nblk = _CPT_AGG // kb

    @functools.partial(
        pl.kernel,
        out_type=jax.ShapeDtypeStruct((2, _NR, d), jnp.float32),
        mesh=_sc_mesh,
        compiler_params=_sc_params,
        scratch_types=[
            pltpu.VMEM((nblk, kb * _CH), jnp.int32),
            pltpu.VMEM((nblk, kb * _CH), jnp.int32),
            pltpu.VMEM((kb * _CH, d), jnp.float32),
            pltpu.VMEM((kb * _CH, d), jnp.float32),
            pltpu.VMEM_SHARED((_NR, d), jnp.float32),
            pltpu.SemaphoreType.DMA,
            pltpu.SemaphoreType.DMA,
        ],
    )
    def agg_kernel(y_hbm, src_hbm, dst_hbm, out_hbm,
                   src_v, dst_v, rows_v, rows1_v, acc_sh, sg0, sg1):
        cid = lax.axis_index("c")
        sid = lax.axis_index("s")
        zero16 = jnp.zeros((16,), jnp.float32)

        pltpu.sync_copy(src_hbm.at[pl.ds(sid * nblk, nblk)], src_v)
        pltpu.sync_copy(dst_hbm.at[pl.ds(sid * nblk, nblk)], dst_v)

        @pl.loop(0, min(_RPTA, kb * _CH))
        def _(r):
            for j in range(d // 16):
                rows_v[r, pl.ds(16 * j, 16)] = zero16

        _nz = kb * _CH
        _off = 0
        while _off < _RPTA:
            _m = min(_nz, _RPTA - _off)
            pltpu.sync_copy(
                rows_v.at[pl.ds(0, _m)],
                acc_sh.at[pl.ds(sid * _RPTA + _off, _m)])
            _off += _m
        plsc.subcore_barrier()

        def _g_start(c, buf, sem):
            pltpu.make_async_copy(
                y_hbm.at[cid].at[src_v.at[c]], buf, sem).start()

        def _g_wait(c, buf, sem):
            pltpu.make_async_copy(
                y_hbm.at[cid].at[src_v.at[c]], buf, sem).wait()

        npair = nblk // 2
        _g_start(0, rows_v, sg0)

        @pl.loop(0, npair)
        def _(p):
            c0 = 2 * p
            _g_start(c0 + 1, rows1_v, sg1)
            _g_wait(c0, rows_v, sg0)
            pltpu.sync_copy(rows_v, acc_sh.at[dst_v.at[c0]], add=True)

            @pl.when(p + 1 < npair)
            def _():
                _g_start(c0 + 2, rows_v, sg0)

            _g_wait(c0 + 1, rows1_v, sg1)
            pltpu.sync_copy(rows1_v, acc_sh.at[dst_v.at[c0 + 1]], add=True)

        plsc.subcore_barrier()
        pltpu.sync_copy(acc_sh.at[pl.ds(sid * _RPTA, _RPTA)],
                        out_hbm.at[cid, pl.ds(sid * _RPTA, _RPTA)])

    return agg_kernel


_sc_degree = _make_sc_degree()
_sc_agg64 = _make_sc_agg(64, _KB)
_sc_agg32 = _make_sc_agg(32, _KB)


def _tc_first(dp, h, w1):
    n, din = h.shape
    dh = w1.shape[1] // 2
    g = n // _BN

    def body(dp_ref, h_ref, w_ref, y_ref, norm_ref):
        deg = jnp.sum(dp_ref[...], axis=1)[:, None] + 1.0
        nrm = lax.rsqrt(jnp.maximum(deg, 1.0))
        norm_ref[...] = nrm
        y = jnp.dot(h_ref[...] * nrm, w_ref[...],
                    preferred_element_type=jnp.float32)
        y_ref[0] = y[:, :dh]
        y_ref[1] = y[:, dh:]

    return pl.pallas_call(
        body,
        grid=(g,),
        in_specs=[
            pl.BlockSpec((_BN, 512), lambda i: (i, 0)),
            pl.BlockSpec((_BN, din), lambda i: (i, 0)),
            pl.BlockSpec((din, 2 * dh), lambda i: (0, 0)),
        ],
        out_specs=[
            pl.BlockSpec((2, _BN, dh), lambda i: (0, i, 0)),
            pl.BlockSpec((_BN, 1), lambda i: (i, 0)),
        ],
        out_shape=[
            jax.ShapeDtypeStruct((2, n, dh), jnp.float32),
            jax.ShapeDtypeStruct((n, 1), jnp.float32),
        ],
    )(dp, h, w1)


def _tc_mid(p, y, norm, b, w):
    n = y.shape[1]
    d = 2 * y.shape[2]
    dout = w.shape[1]
    g = n // _BN

    def body(p_ref, y_ref, n_ref, b_ref, w_ref, o_ref):
        agg = jnp.concatenate(
            [p_ref[0] + y_ref[0], p_ref[1] + y_ref[1]], axis=-1)
        x = jnp.maximum(n_ref[...] * agg + b_ref[...], 0.0)
        yn = jnp.dot(x * n_ref[...], w_ref[...],
                     preferred_element_type=jnp.float32)
        o_ref[0] = yn[:, :dout // 2]
        o_ref[1] = yn[:, dout // 2:]

    return pl.pallas_call(
        body,
        grid=(g,),
        in_specs=[
            pl.BlockSpec((2, _BN, d // 2), lambda i: (0, i, 0)),
            pl.BlockSpec((2, _BN, d // 2), lambda i: (0, i, 0)),
            pl.BlockSpec((_BN, 1), lambda i: (i, 0)),
            pl.BlockSpec((1, d), lambda i: (0, 0)),
            pl.BlockSpec((d, dout), lambda i: (0, 0)),
        ],
        out_specs=pl.BlockSpec((2, _BN, dout // 2), lambda i: (0, i, 0)),
        out_shape=jax.ShapeDtypeStruct((2, n, dout // 2), jnp.float32),
    )(p, y, norm, b, w)


def _tc_last(p, y, norm, b):
    n = y.shape[1]
    d = 2 * y.shape[2]
    g = n // _BN

    def body(p_ref, y_ref, n_ref, b_ref, o_ref):
        agg = jnp.concatenate(
            [p_ref[0] + y_ref[0], p_ref[1] + y_ref[1]], axis=-1)
        o_ref[...] = n_ref[...] * agg + b_ref[...]

    return pl.pallas_call(
        body,
        grid=(g,),
        in_specs=[
            pl.BlockSpec((2, _BN, d // 2), lambda i: (0, i, 0)),
            pl.BlockSpec((2, _BN, d // 2), lambda i: (0, i, 0)),
            pl.BlockSpec((_BN, 1), lambda i: (i, 0)),
            pl.BlockSpec((1, d), lambda i: (0, 0)),
        ],
        out_specs=pl.BlockSpec((_BN, d), lambda i: (i, 0)),
        out_shape=jax.ShapeDtypeStruct((n, d), jnp.float32),
    )(p, y, norm, b)


def kernel(h, edge_index, W1, b1, W2, b2, W3, b3):
    n = h.shape[0]
    src = edge_index[0]
    dst = edge_index[1]
    pad = _E_PAD - src.shape[0]
    src_p = jnp.concatenate([src, jnp.zeros((pad,), jnp.int32)])
    dst_p = jnp.concatenate([dst, jnp.full((pad,), _PAD_DST, jnp.int32)])
    srcA = src_p.reshape(_E_PAD // (_KB * _CH), _KB * _CH)
    dstA = dst_p.reshape(_E_PAD // (_KB * _CH), _KB * _CH)
    srcB = src_p.reshape(_E_PAD // (2 * _KB * _CH), 2 * _KB * _CH)
    dstB = dst_p.reshape(_E_PAD // (2 * _KB * _CH), 2 * _KB * _CH)

    hp = jnp.pad(h, ((0, _NR - n), (0, 0)))

    degp = _sc_degree(dst_p.reshape(_E_PAD // _CH, _CH))

    y1, norm = _tc_first(degp, hp, W1)
    p1 = _sc_agg64(y1, srcA, dstA)
    y2 = _tc_mid(p1, y1, norm, b1.reshape(1, -1), W2)
    p2 = _sc_agg64(y2, srcA, dstA)
    y3 = _tc_mid(p2, y2, norm, b2.reshape(1, -1), W3)
    p3 = _sc_agg32(y3, srcA, dstA)
    return _tc_last(p3, y3, norm, b3.reshape(1, -1))[:n]
